# Initial kernel scaffold; baseline (speedup 1.0000x reference)
import jax
import jax.numpy as jnp
from jax.experimental import pallas as pl


def kernel(x, W):
    raise NotImplementedError("write your pallas kernel here")


# speedup vs baseline: 1.3033x; 1.3033x over previous
import jax
import jax.numpy as jnp
from jax.experimental import pallas as pl
from jax.experimental.pallas import tpu as pltpu

N_EXPERT = 16
DIM_IN = 2048
TILE = 512


def _gating_kernel(x_ref, wt_ref, cw_ref, ei_ref):
    x = x_ref[...]
    wt = wt_ref[...]
    logits = jax.lax.dot_general(
        x, wt, (((1,), (0,)), ((), ())), preferred_element_type=jnp.float32
    )
    t = logits.shape[0]
    iota = jax.lax.broadcasted_iota(jnp.int32, (t, N_EXPERT), 1)

    m1 = jnp.max(logits, axis=-1, keepdims=True)
    idx1 = jnp.min(
        jnp.where(logits == m1, iota, N_EXPERT), axis=-1, keepdims=True
    )
    masked = jnp.where(iota == idx1, -jnp.inf, logits)
    m2 = jnp.max(masked, axis=-1, keepdims=True)
    idx2 = jnp.min(
        jnp.where(masked == m2, iota, N_EXPERT), axis=-1, keepdims=True
    )

    z = jnp.sum(jnp.exp(logits - m1), axis=-1, keepdims=True)
    p1 = 1.0 / z
    p2 = jnp.exp(m2 - m1) / z
    den = p1 + p2 + 1e-09
    cw_ref[:, 0:1] = p1 / den
    cw_ref[:, 1:2] = p2 / den
    ei_ref[:, 0:1] = idx1
    ei_ref[:, 1:2] = idx2


def kernel(x, W):
    b, n, d = x.shape
    tokens = b * n
    xf = x.reshape(tokens, d)
    wt = W.T
    grid = (tokens // TILE,)
    cw, ei = pl.pallas_call(
        _gating_kernel,
        grid=grid,
        in_specs=[
            pl.BlockSpec((TILE, d), lambda i: (i, 0)),
            pl.BlockSpec((d, N_EXPERT), lambda i: (0, 0)),
        ],
        out_specs=[
            pl.BlockSpec((TILE, 2), lambda i: (i, 0)),
            pl.BlockSpec((TILE, 2), lambda i: (i, 0)),
        ],
        out_shape=[
            jax.ShapeDtypeStruct((tokens, 2), jnp.float32),
            jax.ShapeDtypeStruct((tokens, 2), jnp.int32),
        ],
        compiler_params=pltpu.CompilerParams(
            dimension_semantics=("arbitrary",),
        ),
    )(xf, wt)
    return cw.reshape(b, n, 2), ei.reshape(b, n, 2)


# speedup vs baseline: 1.4813x; 1.1366x over previous
import jax
import jax.numpy as jnp
from jax.experimental import pallas as pl
from jax.experimental.pallas import tpu as pltpu

N_EXPERT = 16
DIM_IN = 2048
TILE = 1024


def _gating_kernel(x_ref, wt_ref, cw_ref, ei_ref):
    x = x_ref[...]
    wt = wt_ref[...]
    logits = jax.lax.dot_general(
        x, wt, (((1,), (0,)), ((), ())), preferred_element_type=jnp.float32
    )
    t = logits.shape[0]
    iota = jax.lax.broadcasted_iota(jnp.int32, (t, N_EXPERT), 1)

    m1 = jnp.max(logits, axis=-1, keepdims=True)
    idx1 = jnp.min(
        jnp.where(logits == m1, iota, N_EXPERT), axis=-1, keepdims=True
    )
    masked = jnp.where(iota == idx1, -jnp.inf, logits)
    m2 = jnp.max(masked, axis=-1, keepdims=True)
    idx2 = jnp.min(
        jnp.where(masked == m2, iota, N_EXPERT), axis=-1, keepdims=True
    )

    z = jnp.sum(jnp.exp(logits - m1), axis=-1, keepdims=True)
    p1 = 1.0 / z
    p2 = jnp.exp(m2 - m1) / z
    den = p1 + p2 + 1e-09
    cw_ref[:, 0:1] = p1 / den
    cw_ref[:, 1:2] = p2 / den
    ei_ref[:, 0:1] = idx1
    ei_ref[:, 1:2] = idx2


def kernel(x, W):
    b, n, d = x.shape
    tokens = b * n
    xf = x.reshape(tokens, d)
    wt = W.T
    grid = (tokens // TILE,)
    cw, ei = pl.pallas_call(
        _gating_kernel,
        grid=grid,
        in_specs=[
            pl.BlockSpec((TILE, d), lambda i: (i, 0)),
            pl.BlockSpec((d, N_EXPERT), lambda i: (0, 0)),
        ],
        out_specs=[
            pl.BlockSpec((TILE, 2), lambda i: (i, 0)),
            pl.BlockSpec((TILE, 2), lambda i: (i, 0)),
        ],
        out_shape=[
            jax.ShapeDtypeStruct((tokens, 2), jnp.float32),
            jax.ShapeDtypeStruct((tokens, 2), jnp.int32),
        ],
        compiler_params=pltpu.CompilerParams(
            dimension_semantics=("parallel",),
        ),
    )(xf, wt)
    return cw.reshape(b, n, 2), ei.reshape(b, n, 2)


# speedup vs baseline: 1.4955x; 1.0096x over previous
import jax
import jax.numpy as jnp
from jax.experimental import pallas as pl
from jax.experimental.pallas import tpu as pltpu

N_EXPERT = 16
DIM_IN = 2048
TILE = 2048


def _gating_kernel(x_ref, wt_ref, cw_ref, ei_ref):
    x = x_ref[...]
    wt = wt_ref[...]
    logits = jax.lax.dot_general(
        x, wt, (((1,), (0,)), ((), ())), preferred_element_type=jnp.float32
    )
    t = logits.shape[0]
    iota = jax.lax.broadcasted_iota(jnp.int32, (t, N_EXPERT), 1)

    m1 = jnp.max(logits, axis=-1, keepdims=True)
    idx1 = jnp.min(
        jnp.where(logits == m1, iota, N_EXPERT), axis=-1, keepdims=True
    )
    masked = jnp.where(iota == idx1, -jnp.inf, logits)
    m2 = jnp.max(masked, axis=-1, keepdims=True)
    idx2 = jnp.min(
        jnp.where(masked == m2, iota, N_EXPERT), axis=-1, keepdims=True
    )

    z = jnp.sum(jnp.exp(logits - m1), axis=-1, keepdims=True)
    p1 = 1.0 / z
    p2 = jnp.exp(m2 - m1) / z
    den = p1 + p2 + 1e-09
    cw_ref[:, 0:1] = p1 / den
    cw_ref[:, 1:2] = p2 / den
    ei_ref[:, 0:1] = idx1
    ei_ref[:, 1:2] = idx2


def kernel(x, W):
    b, n, d = x.shape
    tokens = b * n
    xf = x.reshape(tokens, d)
    wt = W.T
    grid = (tokens // TILE,)
    cw, ei = pl.pallas_call(
        _gating_kernel,
        grid=grid,
        in_specs=[
            pl.BlockSpec((TILE, d), lambda i: (i, 0)),
            pl.BlockSpec((d, N_EXPERT), lambda i: (0, 0)),
        ],
        out_specs=[
            pl.BlockSpec((TILE, 2), lambda i: (i, 0)),
            pl.BlockSpec((TILE, 2), lambda i: (i, 0)),
        ],
        out_shape=[
            jax.ShapeDtypeStruct((tokens, 2), jnp.float32),
            jax.ShapeDtypeStruct((tokens, 2), jnp.int32),
        ],
        compiler_params=pltpu.CompilerParams(
            dimension_semantics=("parallel",),
        ),
    )(xf, wt)
    return cw.reshape(b, n, 2), ei.reshape(b, n, 2)
